# baseline (speedup 1.0000x reference)
import functools

import jax
import jax.numpy as jnp
from jax import lax
from jax.experimental import pallas as pl
from jax.experimental.pallas import tpu as pltpu
from jax.experimental.pallas import tpu_sc as plsc

N_NODES = 10000
D = 128

NC = 2
NS = 16
NW = NC * NS
CHUNK = 128
DEG_W = 16

N_PAD = 10240
ROWS_PER_SUB = N_PAD // NS


def _sc_feat_body(x_hbm, src_hbm, dst_hbm, z_hbm, agg_out,
                  src_c, dst_c, rows_v, agg_sh, sem):
    n_chunks = src_hbm.shape[1]
    cid = lax.axis_index("c")
    sid = lax.axis_index("s")
    wid = cid * NS + sid
    row0 = sid * ROWS_PER_SUB
    kch = ROWS_PER_SUB // CHUNK

    pltpu.sync_copy(z_hbm, rows_v)
    for k in range(kch):
        pltpu.sync_copy(rows_v, agg_sh.at[pl.ds(row0 + k * CHUNK, CHUNK)])
    plsc.subcore_barrier()

    def step(j, carry):
        pltpu.sync_copy(src_hbm.at[wid, j], src_c)
        pltpu.sync_copy(dst_hbm.at[wid, j], dst_c)
        pltpu.async_copy(x_hbm.at[src_c], rows_v, sem).wait()
        pltpu.sync_copy(rows_v, agg_sh.at[dst_c], add=True)
        return carry

    lax.fori_loop(0, n_chunks, step, 0)
    plsc.subcore_barrier()
    for k in range(kch):
        pltpu.sync_copy(agg_sh.at[pl.ds(row0 + k * CHUNK, CHUNK)], rows_v)
        pltpu.sync_copy(rows_v, agg_out.at[cid, pl.ds(row0 + k * CHUNK, CHUNK)])


_SC_FEAT = pl.kernel(
    _sc_feat_body,
    out_type=jax.ShapeDtypeStruct((NC, N_PAD, D), jnp.float32),
    mesh=plsc.VectorSubcoreMesh(core_axis_name="c", subcore_axis_name="s"),
    scratch_types=[
        pltpu.VMEM((CHUNK,), jnp.int32),
        pltpu.VMEM((CHUNK,), jnp.int32),
        pltpu.VMEM((CHUNK, D), jnp.float32),
        pltpu.VMEM_SHARED((N_PAD, D), jnp.float32),
        pltpu.SemaphoreType.DMA,
    ],
)

def _tc_linear(agg2, deg2, xin, wlT, wrT, b, relu):
    blk = 400
    grid = (N_NODES // blk,)

    def body(agg_ref, deg_ref, x_ref, wl_ref, wr_ref, b_ref, o_ref):
        agg = agg_ref[0] + agg_ref[1]
        deg = deg_ref[0, :, 0:1] + deg_ref[1, :, 0:1]
        mean = agg * (1.0 / jnp.maximum(deg, 1.0))
        acc = jnp.dot(mean, wl_ref[...], preferred_element_type=jnp.float32)
        acc = acc + jnp.dot(x_ref[...], wr_ref[...],
                            preferred_element_type=jnp.float32)
        acc = acc + b_ref[...]
        if relu:
            acc = jnp.maximum(acc, 0.0)
        o_ref[...] = acc

    return pl.pallas_call(
        body,
        grid=grid,
        in_specs=[
            pl.BlockSpec((NC, blk, D), lambda i: (0, i, 0)),
            pl.BlockSpec((NC, blk, D), lambda i: (0, i, 0)),
            pl.BlockSpec((blk, D), lambda i: (i, 0)),
            pl.BlockSpec((D, D), lambda i: (0, 0)),
            pl.BlockSpec((D, D), lambda i: (0, 0)),
            pl.BlockSpec((1, D), lambda i: (0, 0)),
        ],
        out_specs=pl.BlockSpec((blk, D), lambda i: (i, 0)),
        out_shape=jax.ShapeDtypeStruct((N_NODES, D), jnp.float32),
    )(agg2, deg2, xin, wlT, wrT, b)


def kernel(x, edge_index, W_l1, b_l1, W_r1, W_l2, b_l2, W_r2):
    n_edges = edge_index.shape[1]
    src = edge_index[0].astype(jnp.int32)
    dst = edge_index[1].astype(jnp.int32)

    edges_per_w = -(-n_edges // NW)
    n_chunks = -(-edges_per_w // CHUNK)
    e_pad = NW * n_chunks * CHUNK - n_edges
    src_p = jnp.concatenate(
        [src, jnp.zeros((e_pad,), jnp.int32)]).reshape(NW, n_chunks, CHUNK)
    dst_p = jnp.concatenate(
        [dst, jnp.full((e_pad,), N_NODES, jnp.int32)]).reshape(NW, n_chunks, CHUNK)

    z_agg = jnp.zeros((CHUNK, D), jnp.float32)
    ones_table = jnp.ones((8, D), jnp.float32)
    src_zero = jnp.zeros_like(src_p)

    b1 = b_l1.reshape(1, D)
    b2 = b_l2.reshape(1, D)

    agg1 = _SC_FEAT(x, src_p, dst_p, z_agg)
    deg = _SC_FEAT(ones_table, src_zero, dst_p, z_agg)
    h = _tc_linear(agg1, deg, x, W_l1.T, W_r1.T, b1, relu=True)
    agg2 = _SC_FEAT(h, src_p, dst_p, z_agg)
    out = _tc_linear(agg2, deg, h, W_l2.T, W_r2.T, b2, relu=False)
    return out


# speedup vs baseline: 13.5210x; 13.5210x over previous
import functools

import jax
import jax.numpy as jnp
from jax import lax
from jax.experimental import pallas as pl
from jax.experimental.pallas import tpu as pltpu
from jax.experimental.pallas import tpu_sc as plsc

N_NODES = 10000
D = 128

NC = 2
NS = 16
NW = NC * NS
CHUNK = 128
DEG_W = 16

N_PAD = 10240
ROWS_PER_SUB = N_PAD // NS


def _sc_feat_body(x_hbm, src_hbm, dst_hbm, z_hbm, agg_out,
                  src_c, dst_c, rows_v, agg_sh, sem):
    n_chunks = src_hbm.shape[1]
    cid = lax.axis_index("c")
    sid = lax.axis_index("s")
    wid = cid * NS + sid
    row0 = sid * ROWS_PER_SUB
    kch = ROWS_PER_SUB // CHUNK

    pltpu.sync_copy(z_hbm, rows_v)
    for k in range(kch):
        pltpu.sync_copy(rows_v, agg_sh.at[pl.ds(row0 + k * CHUNK, CHUNK)])
    plsc.subcore_barrier()

    def step(j, carry):
        pltpu.sync_copy(src_hbm.at[wid, j], src_c)
        pltpu.sync_copy(dst_hbm.at[wid, j], dst_c)
        pltpu.async_copy(x_hbm.at[src_c], rows_v, sem).wait()
        pltpu.sync_copy(rows_v, agg_sh.at[dst_c], add=True)
        return carry

    lax.fori_loop(0, n_chunks, step, 0)
    plsc.subcore_barrier()
    for k in range(kch):
        pltpu.sync_copy(agg_sh.at[pl.ds(row0 + k * CHUNK, CHUNK)], rows_v)
        pltpu.sync_copy(rows_v, agg_out.at[cid, pl.ds(row0 + k * CHUNK, CHUNK)])


def _sc_deg_body(dst_hbm, z_hbm, ones_hbm, deg_out, dst_c, rows_v, deg_sh, sem):
    n_chunks = dst_hbm.shape[1]
    cid = lax.axis_index("c")
    sid = lax.axis_index("s")
    wid = cid * NS + sid
    row0 = sid * ROWS_PER_SUB
    kch = ROWS_PER_SUB // CHUNK

    pltpu.sync_copy(z_hbm, rows_v)
    for k in range(kch):
        pltpu.sync_copy(rows_v, deg_sh.at[pl.ds(row0 + k * CHUNK, CHUNK)])
    pltpu.sync_copy(ones_hbm, rows_v)
    plsc.subcore_barrier()

    def step(j, carry):
        pltpu.sync_copy(dst_hbm.at[wid, j], dst_c)
        pltpu.sync_copy(rows_v, deg_sh.at[dst_c], add=True)
        return carry

    lax.fori_loop(0, n_chunks, step, 0)
    plsc.subcore_barrier()
    for k in range(kch):
        pltpu.sync_copy(deg_sh.at[pl.ds(row0 + k * CHUNK, CHUNK)], rows_v)
        pltpu.sync_copy(rows_v, deg_out.at[cid, pl.ds(row0 + k * CHUNK, CHUNK)])


_SC_DEG = pl.kernel(
    _sc_deg_body,
    out_type=jax.ShapeDtypeStruct((NC, N_PAD, D), jnp.float32),
    mesh=plsc.VectorSubcoreMesh(core_axis_name="c", subcore_axis_name="s"),
    scratch_types=[
        pltpu.VMEM((CHUNK,), jnp.int32),
        pltpu.VMEM((CHUNK, D), jnp.float32),
        pltpu.VMEM_SHARED((N_PAD, D), jnp.float32),
        pltpu.SemaphoreType.DMA,
    ],
)


_SC_FEAT = pl.kernel(
    _sc_feat_body,
    out_type=jax.ShapeDtypeStruct((NC, N_PAD, D), jnp.float32),
    mesh=plsc.VectorSubcoreMesh(core_axis_name="c", subcore_axis_name="s"),
    scratch_types=[
        pltpu.VMEM((CHUNK,), jnp.int32),
        pltpu.VMEM((CHUNK,), jnp.int32),
        pltpu.VMEM((CHUNK, D), jnp.float32),
        pltpu.VMEM_SHARED((N_PAD, D), jnp.float32),
        pltpu.SemaphoreType.DMA,
    ],
)

def _tc_linear(agg2, deg2, xin, wlT, wrT, b, relu):
    blk = 400
    grid = (N_NODES // blk,)

    def body(agg_ref, deg_ref, x_ref, wl_ref, wr_ref, b_ref, o_ref):
        agg = agg_ref[0] + agg_ref[1]
        deg = deg_ref[0, :, 0:1] + deg_ref[1, :, 0:1]
        mean = agg * (1.0 / jnp.maximum(deg, 1.0))
        acc = jnp.dot(mean, wl_ref[...], preferred_element_type=jnp.float32)
        acc = acc + jnp.dot(x_ref[...], wr_ref[...],
                            preferred_element_type=jnp.float32)
        acc = acc + b_ref[...]
        if relu:
            acc = jnp.maximum(acc, 0.0)
        o_ref[...] = acc

    return pl.pallas_call(
        body,
        grid=grid,
        in_specs=[
            pl.BlockSpec((NC, blk, D), lambda i: (0, i, 0)),
            pl.BlockSpec((NC, blk, D), lambda i: (0, i, 0)),
            pl.BlockSpec((blk, D), lambda i: (i, 0)),
            pl.BlockSpec((D, D), lambda i: (0, 0)),
            pl.BlockSpec((D, D), lambda i: (0, 0)),
            pl.BlockSpec((1, D), lambda i: (0, 0)),
        ],
        out_specs=pl.BlockSpec((blk, D), lambda i: (i, 0)),
        out_shape=jax.ShapeDtypeStruct((N_NODES, D), jnp.float32),
    )(agg2, deg2, xin, wlT, wrT, b)


def kernel(x, edge_index, W_l1, b_l1, W_r1, W_l2, b_l2, W_r2):
    n_edges = edge_index.shape[1]
    src = edge_index[0].astype(jnp.int32)
    dst = edge_index[1].astype(jnp.int32)

    edges_per_w = -(-n_edges // NW)
    n_chunks = -(-edges_per_w // CHUNK)
    e_pad = NW * n_chunks * CHUNK - n_edges
    src_p = jnp.concatenate(
        [src, jnp.zeros((e_pad,), jnp.int32)]).reshape(NW, n_chunks, CHUNK)
    dst_p = jnp.concatenate(
        [dst, jnp.full((e_pad,), N_NODES, jnp.int32)]).reshape(NW, n_chunks, CHUNK)

    z_agg = jnp.zeros((CHUNK, D), jnp.float32)
    ones_blk = jnp.ones((CHUNK, D), jnp.float32)

    b1 = b_l1.reshape(1, D)
    b2 = b_l2.reshape(1, D)

    agg1 = _SC_FEAT(x, src_p, dst_p, z_agg)
    deg = _SC_DEG(dst_p, z_agg, ones_blk)
    h = _tc_linear(agg1, deg, x, W_l1.T, W_r1.T, b1, relu=True)
    agg2 = _SC_FEAT(h, src_p, dst_p, z_agg)
    out = _tc_linear(agg2, deg, h, W_l2.T, W_r2.T, b2, relu=False)
    return out


# speedup vs baseline: 26.0854x; 1.9293x over previous
import jax
import jax.numpy as jnp
from jax import lax
from jax.experimental import pallas as pl
from jax.experimental.pallas import tpu as pltpu
from jax.experimental.pallas import tpu_sc as plsc

N_NODES = 10000
D = 128

NC = 2
NS = 16
NW = NC * NS
CHUNK = 80
GRP = 5

N_PAD = 10240
ROWS_PER_SUB = N_PAD // NS


def _sc_feat_body(x_hbm, src_hbm, dst_hbm, z_hbm, agg_out,
                  src_g, dst_g, ra_v, rb_v, agg_sh, sem_a, sem_b):
    n_groups = src_hbm.shape[1]
    cid = lax.axis_index("c")
    sid = lax.axis_index("s")
    wid = cid * NS + sid
    row0 = sid * ROWS_PER_SUB

    pltpu.sync_copy(z_hbm, ra_v)
    for k in range(ROWS_PER_SUB // CHUNK):
        pltpu.sync_copy(ra_v, agg_sh.at[pl.ds(row0 + k * CHUNK, CHUNK)])
    plsc.subcore_barrier()

    rows = (ra_v, rb_v)
    sems = (sem_a, sem_b)

    def step(b, carry):
        pltpu.sync_copy(src_hbm.at[wid, b], src_g)
        pltpu.sync_copy(dst_hbm.at[wid, b], dst_g)
        pend = [None, None]
        pend[0] = pltpu.async_copy(x_hbm.at[src_g.at[0]], rows[0], sems[0])
        for g in range(GRP):
            cur, nxt = g % 2, (g + 1) % 2
            if g + 1 < GRP:
                pend[nxt] = pltpu.async_copy(
                    x_hbm.at[src_g.at[g + 1]], rows[nxt], sems[nxt])
            pend[cur].wait()
            pltpu.sync_copy(rows[cur], agg_sh.at[dst_g.at[g]], add=True)
        return carry

    lax.fori_loop(0, n_groups, step, 0)
    plsc.subcore_barrier()
    for k in range(ROWS_PER_SUB // CHUNK):
        pltpu.sync_copy(agg_sh.at[pl.ds(row0 + k * CHUNK, CHUNK)], ra_v)
        pltpu.sync_copy(ra_v, agg_out.at[cid, pl.ds(row0 + k * CHUNK, CHUNK)])


def _sc_deg_body(dst_hbm, z_hbm, ones_hbm, deg_out, dst_g, ones_v, deg_sh, sem):
    n_groups = dst_hbm.shape[1]
    cid = lax.axis_index("c")
    sid = lax.axis_index("s")
    wid = cid * NS + sid
    row0 = sid * ROWS_PER_SUB

    pltpu.sync_copy(z_hbm, ones_v)
    for k in range(ROWS_PER_SUB // CHUNK):
        pltpu.sync_copy(ones_v, deg_sh.at[pl.ds(row0 + k * CHUNK, CHUNK)])
    pltpu.sync_copy(ones_hbm, ones_v)
    plsc.subcore_barrier()

    def step(b, carry):
        pltpu.sync_copy(dst_hbm.at[wid, b], dst_g)
        for g in range(GRP):
            pltpu.sync_copy(ones_v, deg_sh.at[dst_g.at[g]], add=True)
        return carry

    lax.fori_loop(0, n_groups, step, 0)
    plsc.subcore_barrier()
    for k in range(ROWS_PER_SUB // CHUNK):
        pltpu.sync_copy(deg_sh.at[pl.ds(row0 + k * CHUNK, CHUNK)], ones_v)
        pltpu.sync_copy(ones_v, deg_out.at[cid, pl.ds(row0 + k * CHUNK, CHUNK)])


_SC_FEAT = pl.kernel(
    _sc_feat_body,
    out_type=jax.ShapeDtypeStruct((NC, N_PAD, D), jnp.float32),
    mesh=plsc.VectorSubcoreMesh(core_axis_name="c", subcore_axis_name="s"),
    scratch_types=[
        pltpu.VMEM((GRP, CHUNK), jnp.int32),
        pltpu.VMEM((GRP, CHUNK), jnp.int32),
        pltpu.VMEM((CHUNK, D), jnp.float32),
        pltpu.VMEM((CHUNK, D), jnp.float32),
        pltpu.VMEM_SHARED((N_PAD, D), jnp.float32),
        pltpu.SemaphoreType.DMA,
        pltpu.SemaphoreType.DMA,
    ],
)

_SC_DEG = pl.kernel(
    _sc_deg_body,
    out_type=jax.ShapeDtypeStruct((NC, N_PAD, D), jnp.float32),
    mesh=plsc.VectorSubcoreMesh(core_axis_name="c", subcore_axis_name="s"),
    scratch_types=[
        pltpu.VMEM((GRP, CHUNK), jnp.int32),
        pltpu.VMEM((CHUNK, D), jnp.float32),
        pltpu.VMEM_SHARED((N_PAD, D), jnp.float32),
        pltpu.SemaphoreType.DMA,
    ],
)


def _tc_linear(agg2, deg2, xin, wlT, wrT, b, relu):
    blk = 400
    grid = (N_NODES // blk,)

    def body(agg_ref, deg_ref, x_ref, wl_ref, wr_ref, b_ref, o_ref):
        agg = agg_ref[0] + agg_ref[1]
        deg = deg_ref[0, :, 0:1] + deg_ref[1, :, 0:1]
        mean = agg * (1.0 / jnp.maximum(deg, 1.0))
        acc = jnp.dot(mean, wl_ref[...], preferred_element_type=jnp.float32)
        acc = acc + jnp.dot(x_ref[...], wr_ref[...],
                            preferred_element_type=jnp.float32)
        acc = acc + b_ref[...]
        if relu:
            acc = jnp.maximum(acc, 0.0)
        o_ref[...] = acc

    return pl.pallas_call(
        body,
        grid=grid,
        in_specs=[
            pl.BlockSpec((NC, blk, D), lambda i: (0, i, 0)),
            pl.BlockSpec((NC, blk, D), lambda i: (0, i, 0)),
            pl.BlockSpec((blk, D), lambda i: (i, 0)),
            pl.BlockSpec((D, D), lambda i: (0, 0)),
            pl.BlockSpec((D, D), lambda i: (0, 0)),
            pl.BlockSpec((1, D), lambda i: (0, 0)),
        ],
        out_specs=pl.BlockSpec((blk, D), lambda i: (i, 0)),
        out_shape=jax.ShapeDtypeStruct((N_NODES, D), jnp.float32),
    )(agg2, deg2, xin, wlT, wrT, b)


def kernel(x, edge_index, W_l1, b_l1, W_r1, W_l2, b_l2, W_r2):
    n_edges = edge_index.shape[1]
    src = edge_index[0].astype(jnp.int32)
    dst = edge_index[1].astype(jnp.int32)

    per_w = NW * GRP * CHUNK
    n_groups = -(-n_edges // per_w)
    e_pad = NW * n_groups * GRP * CHUNK - n_edges
    src_p = jnp.concatenate(
        [src, jnp.zeros((e_pad,), jnp.int32)]).reshape(NW, n_groups, GRP, CHUNK)
    dst_p = jnp.concatenate(
        [dst, jnp.full((e_pad,), N_NODES, jnp.int32)]).reshape(
            NW, n_groups, GRP, CHUNK)

    z_blk = jnp.zeros((CHUNK, D), jnp.float32)
    ones_blk = jnp.ones((CHUNK, D), jnp.float32)

    b1 = b_l1.reshape(1, D)
    b2 = b_l2.reshape(1, D)

    agg1 = _SC_FEAT(x, src_p, dst_p, z_blk)
    deg = _SC_DEG(dst_p, z_blk, ones_blk)
    h = _tc_linear(agg1, deg, x, W_l1.T, W_r1.T, b1, relu=True)
    agg2 = _SC_FEAT(h, src_p, dst_p, z_blk)
    out = _tc_linear(agg2, deg, h, W_l2.T, W_r2.T, b2, relu=False)
    return out
